# baseline (speedup 1.0000x reference)
import functools

import jax
import jax.numpy as jnp
from jax import lax
from jax.experimental import pallas as pl
from jax.experimental.pallas import tpu as pltpu
from jax.experimental.pallas import tpu_sc as plsc


def _make_sc_gather(B, T, I, E, K):
    N = E * B * K
    NW = 32
    CH = N // NW
    mesh = plsc.VectorSubcoreMesh(core_axis_name="c", subcore_axis_name="s")

    @functools.partial(
        pl.kernel,
        out_type=jax.ShapeDtypeStruct((N,), jnp.float32),
        mesh=mesh,
        scratch_types=[
            pltpu.VMEM((CH,), jnp.int32),
            pltpu.VMEM((CH,), jnp.float32),
            pltpu.SemaphoreType.DMA,
        ],
    )
    def sc_gather(x_hbm, ind_hbm, out_hbm, idx_v, val_v, sem):
        wid = lax.axis_index("s") * 2 + lax.axis_index("c")
        base = wid * CH
        pltpu.sync_copy(ind_hbm.at[pl.ds(base, CH)], idx_v)
        for i in range(CH // 16):
            p0 = base + i * 16
            b = (p0 // K) % B
            vec = idx_v[pl.ds(i * 16, 16)]
            idx_v[pl.ds(i * 16, 16)] = (vec + b * T) * I
        copies = []
        for c in range(CH // 128):
            copies.append(
                pltpu.async_copy(
                    x_hbm.at[idx_v.at[pl.ds(c * 128, 128)]],
                    val_v.at[pl.ds(c * 128, 128)],
                    sem,
                )
            )
        for cp in copies:
            cp.wait()
        pltpu.sync_copy(val_v, out_hbm.at[pl.ds(base, CH)])

    return sc_gather


def _make_tc_mult(B, E, K, I, J, IB):
    NI = I // IB

    def body(xg_ref, w_ref, y_ref, acc_ref):
        i = pl.program_id(1)
        part = jnp.sum(w_ref[0], axis=0, keepdims=True)

        @pl.when(i == 0)
        def _():
            acc_ref[...] = part

        @pl.when(i > 0)
        def _():
            acc_ref[...] += part

        @pl.when(i == NI - 1)
        def _():
            wsum = acc_ref[...]
            xg = xg_ref[0]
            y_ref[...] = xg[:, None, :, None] * wsum.reshape(1, 1, 1, J)

    return pl.pallas_call(
        body,
        grid=(E, NI),
        in_specs=[
            pl.BlockSpec((1, B, K), lambda e, i: (e, 0, 0)),
            pl.BlockSpec((1, IB, J), lambda e, i: (e, i, 0)),
        ],
        out_specs=pl.BlockSpec((B, 1, K, J), lambda e, i: (0, e, 0, 0)),
        out_shape=jax.ShapeDtypeStruct((B, E, K, J), jnp.float32),
        scratch_shapes=[pltpu.VMEM((1, J), jnp.float32)],
        compiler_params=pltpu.CompilerParams(
            dimension_semantics=("arbitrary", "arbitrary"),
        ),
    )


def kernel(X, ind, W):
    B, T, I = X.shape
    E, K = ind.shape[1], ind.shape[2]
    J = W.shape[2]

    ind_flat = jnp.transpose(ind, (1, 0, 2)).reshape(-1)
    xg_flat = _make_sc_gather(B, T, I, E, K)(X.reshape(-1), ind_flat)
    xg = xg_flat.reshape(E, B, K)

    return _make_tc_mult(B, E, K, I, J, IB=256)(xg, W)


# speedup vs baseline: 1.2999x; 1.2999x over previous
import functools

import jax
import jax.numpy as jnp
from jax import lax
from jax.experimental import pallas as pl
from jax.experimental.pallas import tpu as pltpu
from jax.experimental.pallas import tpu_sc as plsc


def _make_sc_gather(B, T, I, E, K):
    N = B * E * K
    NW = 32
    CH = N // NW
    EK = E * K
    mesh = plsc.VectorSubcoreMesh(core_axis_name="c", subcore_axis_name="s")

    @functools.partial(
        pl.kernel,
        out_type=jax.ShapeDtypeStruct((N,), jnp.float32),
        mesh=mesh,
        scratch_types=[
            pltpu.VMEM((CH,), jnp.int32),
            pltpu.VMEM((CH,), jnp.float32),
            pltpu.SemaphoreType.DMA,
        ],
    )
    def sc_gather(x_hbm, ind_hbm, out_hbm, idx_v, val_v, sem):
        wid = lax.axis_index("s") * 2 + lax.axis_index("c")
        base = wid * CH
        pltpu.sync_copy(ind_hbm.at[pl.ds(base, CH)], idx_v)
        b = base // EK
        for i in range(CH // 16):
            vec = idx_v[pl.ds(i * 16, 16)]
            idx_v[pl.ds(i * 16, 16)] = (vec + b * T) * I
        copies = []
        for c in range(CH // 128):
            copies.append(
                pltpu.async_copy(
                    x_hbm.at[idx_v.at[pl.ds(c * 128, 128)]],
                    val_v.at[pl.ds(c * 128, 128)],
                    sem,
                )
            )
        for cp in copies:
            cp.wait()
        pltpu.sync_copy(val_v, out_hbm.at[pl.ds(base, CH)])

    return sc_gather


def _make_tc_mult(B, E, K, I, J):
    def body(xg_ref, w_ref, y_ref):
        wsum = jnp.sum(w_ref[0], axis=0, keepdims=True)
        xg = xg_ref[...]
        y_ref[...] = xg[:, None, :, None] * wsum.reshape(1, 1, 1, J)

    return pl.pallas_call(
        body,
        grid=(E,),
        in_specs=[
            pl.BlockSpec((B, K), lambda e: (0, e)),
            pl.BlockSpec((1, I, J), lambda e: (e, 0, 0)),
        ],
        out_specs=pl.BlockSpec((B, 1, K, J), lambda e: (0, e, 0, 0)),
        out_shape=jax.ShapeDtypeStruct((B, E, K, J), jnp.float32),
        compiler_params=pltpu.CompilerParams(
            dimension_semantics=("arbitrary",),
        ),
    )


def kernel(X, ind, W):
    B, T, I = X.shape
    E, K = ind.shape[1], ind.shape[2]
    J = W.shape[2]

    xg_flat = _make_sc_gather(B, T, I, E, K)(X.reshape(-1), ind.reshape(-1))
    xg = xg_flat.reshape(B, E * K)

    return _make_tc_mult(B, E, K, I, J)(xg, W)


# speedup vs baseline: 1.8338x; 1.4107x over previous
import functools

import jax
import jax.numpy as jnp
from jax import lax
from jax.experimental import pallas as pl
from jax.experimental.pallas import tpu as pltpu
from jax.experimental.pallas import tpu_sc as plsc


def _make_sc_gather(B, T, E, K):
    N = B * E * K
    NW = 32
    CH = N // NW
    EK = E * K
    mesh = plsc.VectorSubcoreMesh(core_axis_name="c", subcore_axis_name="s")

    @functools.partial(
        pl.kernel,
        out_type=jax.ShapeDtypeStruct((N,), jnp.float32),
        mesh=mesh,
        scratch_types=[
            pltpu.VMEM((CH,), jnp.int32),
            pltpu.VMEM((CH,), jnp.float32),
            pltpu.SemaphoreType.DMA,
        ],
    )
    def sc_gather(x0_hbm, ind_hbm, out_hbm, idx_v, val_v, sem):
        wid = lax.axis_index("s") * 2 + lax.axis_index("c")
        base = wid * CH
        pltpu.sync_copy(ind_hbm.at[pl.ds(base, CH)], idx_v)
        b = base // EK
        for i in range(CH // 16):
            vec = idx_v[pl.ds(i * 16, 16)]
            idx_v[pl.ds(i * 16, 16)] = vec + b * T
        copies = []
        for c in range(CH // 128):
            copies.append(
                pltpu.async_copy(
                    x0_hbm.at[idx_v.at[pl.ds(c * 128, 128)]],
                    val_v.at[pl.ds(c * 128, 128)],
                    sem,
                )
            )
        for cp in copies:
            cp.wait()
        pltpu.sync_copy(val_v, out_hbm.at[pl.ds(base, CH)])

    return sc_gather


def _make_tc_reduce(E, I, J):
    def body(w_ref, ws_ref):
        ws_ref[...] = jnp.sum(w_ref[0], axis=0, keepdims=True)[None]

    return pl.pallas_call(
        body,
        grid=(E,),
        in_specs=[pl.BlockSpec((1, I, J), lambda e: (e, 0, 0))],
        out_specs=pl.BlockSpec((1, 1, J), lambda e: (e, 0, 0)),
        out_shape=jax.ShapeDtypeStruct((E, 1, J), jnp.float32),
        compiler_params=pltpu.CompilerParams(
            dimension_semantics=("arbitrary",),
        ),
    )


def _make_tc_broadcast(B, E, K, J):
    def body(xg_ref, ws_ref, y_ref):
        xg = xg_ref[...]
        y_ref[...] = xg[:, None, :, None] * ws_ref[...].reshape(1, 1, 1, J)

    return pl.pallas_call(
        body,
        grid=(E,),
        in_specs=[
            pl.BlockSpec((B, K), lambda e: (0, e)),
            pl.BlockSpec((1, 1, J), lambda e: (e, 0, 0)),
        ],
        out_specs=pl.BlockSpec((B, 1, K, J), lambda e: (0, e, 0, 0)),
        out_shape=jax.ShapeDtypeStruct((B, E, K, J), jnp.float32),
        compiler_params=pltpu.CompilerParams(
            dimension_semantics=("arbitrary",),
        ),
    )


def kernel(X, ind, W):
    B, T, I = X.shape
    E, K = ind.shape[1], ind.shape[2]
    J = W.shape[2]

    x0 = X[:, :, 0].reshape(-1)
    xg_flat = _make_sc_gather(B, T, E, K)(x0, ind.reshape(-1))
    xg = xg_flat.reshape(B, E * K)

    wsum = _make_tc_reduce(E, I, J)(W)
    return _make_tc_broadcast(B, E, K, J)(xg, wsum)


# speedup vs baseline: 1.9170x; 1.0454x over previous
import functools

import jax
import jax.numpy as jnp
from jax import lax
from jax.experimental import pallas as pl
from jax.experimental.pallas import tpu as pltpu
from jax.experimental.pallas import tpu_sc as plsc


def _make_sc_gather(B, T, E, K):
    N = B * E * K
    NW = 32
    CH = N // NW
    EK = E * K
    mesh = plsc.VectorSubcoreMesh(core_axis_name="c", subcore_axis_name="s")

    @functools.partial(
        pl.kernel,
        out_type=jax.ShapeDtypeStruct((N,), jnp.float32),
        mesh=mesh,
        scratch_types=[
            pltpu.VMEM((CH,), jnp.int32),
            pltpu.VMEM((CH,), jnp.float32),
            pltpu.SemaphoreType.DMA,
        ],
    )
    def sc_gather(x0_hbm, ind_hbm, out_hbm, idx_v, val_v, sem):
        wid = lax.axis_index("s") * 2 + lax.axis_index("c")
        base = wid * CH
        pltpu.sync_copy(ind_hbm.at[pl.ds(base, CH)], idx_v)
        b = base // EK
        for i in range(CH // 16):
            vec = idx_v[pl.ds(i * 16, 16)]
            idx_v[pl.ds(i * 16, 16)] = vec + b * T
        copies = []
        for c in range(CH // 128):
            copies.append(
                pltpu.async_copy(
                    x0_hbm.at[idx_v.at[pl.ds(c * 128, 128)]],
                    val_v.at[pl.ds(c * 128, 128)],
                    sem,
                )
            )
        for cp in copies:
            cp.wait()
        pltpu.sync_copy(val_v, out_hbm.at[pl.ds(base, CH)])

    return sc_gather


def _make_tc_fused(B, E, K, I, J):
    def body(xg_ref, w_ref, y_ref):
        wsum = jnp.sum(w_ref[0], axis=0, keepdims=True)
        xg = xg_ref[...]
        y_ref[...] = xg[:, None, :, None] * wsum.reshape(1, 1, 1, J)

    return pl.pallas_call(
        body,
        grid=(E,),
        in_specs=[
            pl.BlockSpec((B, K), lambda e: (0, e)),
            pl.BlockSpec((1, I, J), lambda e: (e, 0, 0)),
        ],
        out_specs=pl.BlockSpec((B, 1, K, J), lambda e: (0, e, 0, 0)),
        out_shape=jax.ShapeDtypeStruct((B, E, K, J), jnp.float32),
        compiler_params=pltpu.CompilerParams(
            dimension_semantics=("arbitrary",),
        ),
    )


def _make_tc_reduce(E, I, J):
    def body(w_ref, ws_ref):
        ws_ref[...] = jnp.sum(w_ref[0], axis=0, keepdims=True)[None]

    return pl.pallas_call(
        body,
        grid=(E,),
        in_specs=[pl.BlockSpec((1, I, J), lambda e: (e, 0, 0))],
        out_specs=pl.BlockSpec((1, 1, J), lambda e: (e, 0, 0)),
        out_shape=jax.ShapeDtypeStruct((E, 1, J), jnp.float32),
        compiler_params=pltpu.CompilerParams(
            dimension_semantics=("arbitrary",),
        ),
    )


def _make_tc_broadcast(B, E, K, J):
    def body(xg_ref, ws_ref, y_ref):
        xg = xg_ref[...]
        y_ref[...] = xg[:, None, :, None] * ws_ref[...].reshape(1, 1, 1, J)

    return pl.pallas_call(
        body,
        grid=(E,),
        in_specs=[
            pl.BlockSpec((B, K), lambda e: (0, e)),
            pl.BlockSpec((1, 1, J), lambda e: (e, 0, 0)),
        ],
        out_specs=pl.BlockSpec((B, 1, K, J), lambda e: (0, e, 0, 0)),
        out_shape=jax.ShapeDtypeStruct((B, E, K, J), jnp.float32),
        compiler_params=pltpu.CompilerParams(
            dimension_semantics=("arbitrary",),
        ),
    )


def kernel(X, ind, W):
    B, T, I = X.shape
    E, K = ind.shape[1], ind.shape[2]
    J = W.shape[2]

    x0 = X[:, :, 0].reshape(-1)
    xg_flat = _make_sc_gather(B, T, E, K)(x0, ind.reshape(-1))
    xg = xg_flat.reshape(B, E * K)

    return _make_tc_fused(B, E, K, I, J)(xg, W)
